# Initial kernel scaffold; baseline (speedup 1.0000x reference)
import jax
import jax.numpy as jnp
from jax.experimental import pallas as pl


def kernel(proto0, proto1, gumbel_u, W_init, b_init, W_ih, b_ih, W_hh, b_hh, W_out, b_out, E):
    raise NotImplementedError("write your pallas kernel here")


# speedup vs baseline: 6.1849x; 6.1849x over previous
import jax
import jax.numpy as jnp
from jax.experimental import pallas as pl
from jax.experimental.pallas import tpu as pltpu

_R = 512
_E_TOK = 256
_H = 512
_V = 1024
_VF = _V + 2
_MSG_LEN = 16
_TAU = 1.0
_UNIFORM_W = 0.1
_SOS_IDX = 0
_EOS_IDX = 1
_EPS = 1e-10

_BB = 512


def _body(p0_ref, p1_ref, u_ref, Wi_ref, bi_ref, Wih_ref, bih_ref,
          Whh_ref, bhh_ref, Wout_ref, bout_ref, E_ref,
          out_ref, h_ref, x_ref):
    t = pl.program_id(1)
    lane = jax.lax.broadcasted_iota(jnp.int32, (_BB, _VF), 1)

    @pl.when(t == 0)
    def _init():
        pp = jnp.concatenate([p0_ref[...], p1_ref[...]], axis=1)
        h_ref[...] = jnp.dot(pp, Wi_ref[...],
                             preferred_element_type=jnp.float32) + bi_ref[...]
        x_ref[...] = jnp.broadcast_to(E_ref[_SOS_IDX:_SOS_IDX + 1, :],
                                      (_BB, _E_TOK))
        out_ref[:, 0, 0, :] = (lane == _SOS_IDX).astype(jnp.float32)

    @pl.when((t >= 1) & (t <= _MSG_LEN - 2))
    def _step():
        h = h_ref[...]
        x = x_ref[...]
        gi = jnp.dot(x, Wih_ref[...],
                     preferred_element_type=jnp.float32) + bih_ref[...]
        gh = jnp.dot(h, Whh_ref[...],
                     preferred_element_type=jnp.float32) + bhh_ref[...]
        r = jax.nn.sigmoid(gi[:, :_H] + gh[:, :_H])
        z = jax.nn.sigmoid(gi[:, _H:2 * _H] + gh[:, _H:2 * _H])
        n = jnp.tanh(gi[:, 2 * _H:] + r * gh[:, 2 * _H:])
        h_new = (1.0 - z) * n + z * h
        h_ref[...] = h_new

        logits = jnp.dot(h_new, Wout_ref[...],
                         preferred_element_type=jnp.float32) + bout_ref[...]
        u = u_ref[0]
        g = -jnp.log(-jnp.log(u + _EPS) + _EPS)
        s = (logits + g) / _TAU
        m = jnp.max(s, axis=-1, keepdims=True)
        p = jnp.exp(s - m)
        y = p / jnp.sum(p, axis=-1, keepdims=True)
        y = (1.0 - _UNIFORM_W) * y + _UNIFORM_W / _V
        idx = jnp.argmax(y, axis=-1, keepdims=True)
        onehot = (lane == idx).astype(jnp.float32)
        st = onehot - y + y
        out_ref[:, 0, 0, :] = st
        x_ref[...] = jnp.dot(st, E_ref[...],
                             preferred_element_type=jnp.float32)

    @pl.when(t == _MSG_LEN - 1)
    def _eos():
        out_ref[:, 0, 0, :] = (lane == _EOS_IDX).astype(jnp.float32)


def kernel(proto0, proto1, gumbel_u, W_init, b_init, W_ih, b_ih, W_hh, b_hh,
           W_out, b_out, E):
    B = proto0.shape[0]
    nb = B // _BB
    grid = (nb, _MSG_LEN)

    Wi_T = W_init.T
    Wih_T = W_ih.T
    Whh_T = W_hh.T
    Wout_T = W_out.T
    bi2 = b_init.reshape(1, _H)
    bih2 = b_ih.reshape(1, 3 * _H)
    bhh2 = b_hh.reshape(1, 3 * _H)
    bout2 = b_out.reshape(1, _VF)

    full = lambda shape: pl.BlockSpec(shape, lambda b, t: (0,) * len(shape))
    out = pl.pallas_call(
        _body,
        grid=grid,
        in_specs=[
            pl.BlockSpec((_BB, _R), lambda b, t: (b, 0)),
            pl.BlockSpec((_BB, _R), lambda b, t: (b, 0)),
            pl.BlockSpec((1, _BB, _VF),
                         lambda b, t: (jnp.clip(t - 1, 0, _MSG_LEN - 3), b, 0)),
            full((2 * _R, _H)),
            full((1, _H)),
            full((_E_TOK, 3 * _H)),
            full((1, 3 * _H)),
            full((_H, 3 * _H)),
            full((1, 3 * _H)),
            full((_H, _VF)),
            full((1, _VF)),
            full((_VF, _E_TOK)),
        ],
        out_specs=pl.BlockSpec((_BB, 1, 1, _VF), lambda b, t: (b, t, 0, 0)),
        out_shape=jax.ShapeDtypeStruct((B, _MSG_LEN, 1, _VF), jnp.float32),
        scratch_shapes=[
            pltpu.VMEM((_BB, _H), jnp.float32),
            pltpu.VMEM((_BB, _E_TOK), jnp.float32),
        ],
        compiler_params=pltpu.CompilerParams(
            dimension_semantics=("parallel", "arbitrary"),
            vmem_limit_bytes=100 * 1024 * 1024,
        ),
        name="sender_gru_lm",
    )(proto0, proto1, gumbel_u, Wi_T, bi2, Wih_T, bih2, Whh_T, bhh2,
      Wout_T, bout2, E)
    return out.reshape(B, _MSG_LEN, _VF)
